# baseline (speedup 1.0000x reference)
import functools

import jax
import jax.numpy as jnp
from jax import lax
from jax.experimental import pallas as pl
from jax.experimental.pallas import tpu as pltpu
from jax.experimental.pallas import tpu_sc as plsc

L = 16
IDX_CHUNK = 128


def _lane_shuffle(v, perm):
    dnums = lax.GatherDimensionNumbers(
        offset_dims=(), collapsed_slice_dims=(0,), start_index_map=(0,))
    return lax.gather(v, perm.reshape(L, 1), dnums, slice_sizes=(1,),
                      mode=lax.GatherScatterMode.PROMISE_IN_BOUNDS)


def _neg_sqrt(s):
    i = lax.bitcast_convert_type(s, jnp.int32)
    i = jnp.int32(0x5F3759DF) - lax.shift_right_logical(i, 1)
    y = lax.bitcast_convert_type(i, jnp.float32)
    half_s = s * jnp.float32(0.5)
    for _ in range(3):
        y = y * (jnp.float32(1.5) - half_s * y * y)
    return -(s * y)


def _make_kernel(B, D, NC, NS):
    NW = NC * NS
    b_w = B // NW
    n_chunks = b_w // IDX_CHUNK
    n_groups = b_w // L
    d_chunks = D // L

    mesh = plsc.VectorSubcoreMesh(core_axis_name="c", subcore_axis_name="s")

    @functools.partial(
        pl.kernel,
        mesh=mesh,
        compiler_params=pltpu.CompilerParams(use_tc_tiling_on_sc=False),
        out_type=jax.ShapeDtypeStruct((B,), jnp.float32),
        scratch_types=[
            pltpu.VMEM((n_chunks, IDX_CHUNK), jnp.int32),
            pltpu.VMEM((n_chunks, IDX_CHUNK), jnp.int32),
            pltpu.VMEM((n_chunks, IDX_CHUNK), jnp.int32),
            pltpu.VMEM((b_w, D), jnp.float32),
            pltpu.VMEM((b_w, D), jnp.float32),
            pltpu.VMEM((b_w, D), jnp.float32),
            pltpu.VMEM((L * L,), jnp.float32),
            pltpu.VMEM((b_w,), jnp.float32),
            pltpu.SemaphoreType.DMA,
        ],
    )
    def kge_kernel(head_hbm, rel_hbm, tail_hbm, node_hbm, relemb_hbm,
                   out_hbm, hidx, ridx, tidx, h_rows, r_rows, t_rows,
                   accs, out_v, sem):
        wid = lax.axis_index("s") * NC + lax.axis_index("c")
        base = wid * b_w

        for j in range(n_chunks):
            off = base + j * IDX_CHUNK
            pltpu.sync_copy(head_hbm.at[pl.ds(off, IDX_CHUNK)], hidx.at[j])
            pltpu.sync_copy(rel_hbm.at[pl.ds(off, IDX_CHUNK)], ridx.at[j])
            pltpu.sync_copy(tail_hbm.at[pl.ds(off, IDX_CHUNK)], tidx.at[j])

        copies = []
        for j in range(n_chunks):
            rs = pl.ds(j * IDX_CHUNK, IDX_CHUNK)
            copies.append(pltpu.async_copy(
                node_hbm.at[hidx.at[j]], h_rows.at[rs], sem))
            copies.append(pltpu.async_copy(
                relemb_hbm.at[ridx.at[j]], r_rows.at[rs], sem))
            copies.append(pltpu.async_copy(
                node_hbm.at[tidx.at[j]], t_rows.at[rs], sem))
        for c in copies:
            c.wait()

        lane_iota = lax.iota(jnp.int32, L)

        def group_body(g, carry):
            rb = g * L
            for r in range(L):
                row = rb + r
                acc = None
                for c in range(d_chunks):
                    cs = pl.ds(c * L, L)
                    d = h_rows[row, cs] + r_rows[row, cs] - t_rows[row, cs]
                    acc = d * d if acc is None else acc + d * d
                for step in (8, 4, 2, 1):
                    acc = acc + _lane_shuffle(acc, lane_iota ^ step)
                sel = lane_iota == jnp.int32(r)
                tot = acc if r == 0 else jnp.where(sel, acc, tot)
            out_v[pl.ds(rb, L)] = _neg_sqrt(tot + jnp.float32(1e-12))
            return carry

        lax.fori_loop(0, n_groups, group_body, 0)
        pltpu.sync_copy(out_v, out_hbm.at[pl.ds(base, b_w)])

    return kge_kernel


def kernel(head_index, rel_type, tail_index, node_emb, rel_emb):
    B = head_index.shape[0]
    D = node_emb.shape[1]
    info = plsc.get_sparse_core_info()
    k = _make_kernel(B, D, info.num_cores, info.num_subcores)
    return k(head_index.astype(jnp.int32), rel_type.astype(jnp.int32),
             tail_index.astype(jnp.int32), node_emb, rel_emb)
